# baseline (speedup 1.0000x reference)
import functools

import jax
import jax.numpy as jnp
from jax import lax
from jax.experimental import pallas as pl
from jax.experimental.pallas import tpu as pltpu, tpu_sc as plsc

NUM_NODES = 10000
MEMORY_DIM = 64
MSG_DIM = 64
NODE_DIM = 256
EDGE_DIM = 16
TIME_DIM = 16
STATIC_DIM = 64
DYNAMIC_DIM = 64
EMB_DIM = 32
B = 2048
RAW_MSG_DIM = EDGE_DIM + TIME_DIM

NC = 2
NS = 16
NW = NC * NS
BPW = B // NW
NODES_PAD = 10016


def _sc_gather_body(mem_hbm, emb_hbm, src_hbm, dst_hbm,
                    o_sm, o_dm, o_es, o_ed,
                    idx_s, idx_d, r_sm, r_dm, r_es, r_ed, sem):
    wid = lax.axis_index("s") * NC + lax.axis_index("c")
    base = wid * BPW
    pltpu.sync_copy(src_hbm.at[pl.ds(base, BPW)], idx_s)
    pltpu.sync_copy(dst_hbm.at[pl.ds(base, BPW)], idx_d)
    c1 = pltpu.async_copy(mem_hbm.at[idx_s], r_sm, sem)
    c2 = pltpu.async_copy(mem_hbm.at[idx_d], r_dm, sem)
    c3 = pltpu.async_copy(emb_hbm.at[idx_s], r_es, sem)
    c4 = pltpu.async_copy(emb_hbm.at[idx_d], r_ed, sem)
    c1.wait(); c2.wait(); c3.wait(); c4.wait()
    pltpu.sync_copy(r_sm, o_sm.at[pl.ds(base, BPW)])
    pltpu.sync_copy(r_dm, o_dm.at[pl.ds(base, BPW)])
    pltpu.sync_copy(r_es, o_es.at[pl.ds(base, BPW)])
    pltpu.sync_copy(r_ed, o_ed.at[pl.ds(base, BPW)])


def _sc_gather(memory, emb_table, src_i, dst_i):
    mesh = plsc.VectorSubcoreMesh(core_axis_name="c", subcore_axis_name="s")
    f32 = jnp.float32
    run = pl.kernel(
        _sc_gather_body,
        out_type=[
            jax.ShapeDtypeStruct((B, MEMORY_DIM), f32),
            jax.ShapeDtypeStruct((B, MEMORY_DIM), f32),
            jax.ShapeDtypeStruct((B, EMB_DIM), f32),
            jax.ShapeDtypeStruct((B, EMB_DIM), f32),
        ],
        mesh=mesh,
        scratch_types=[
            pltpu.VMEM((BPW,), jnp.int32),
            pltpu.VMEM((BPW,), jnp.int32),
            pltpu.VMEM((BPW, MEMORY_DIM), f32),
            pltpu.VMEM((BPW, MEMORY_DIM), f32),
            pltpu.VMEM((BPW, EMB_DIM), f32),
            pltpu.VMEM((BPW, EMB_DIM), f32),
            pltpu.SemaphoreType.DMA,
        ],
        compiler_params=pltpu.CompilerParams(use_tc_tiling_on_sc=False),
    )
    return run(memory, emb_table, src_i, dst_i)


TB = 2048
GRID1 = B // TB


def _tc_msg_body(tf_ref, ea_ref, sm_ref, dm_ref, wt_ref, bt_ref,
                 w1_ref, b1_ref, eet_ref, bbt_ref, o_ref, a_ref):
    f32 = jnp.float32
    tf = tf_ref[...]
    te = jnp.cos(tf * wt_ref[...] + bt_ref[...])
    raw = jnp.concatenate([ea_ref[...], te], axis=1)
    h = jnp.maximum(
        jnp.dot(raw, w1_ref[...], preferred_element_type=f32)
        + b1_ref[...], 0.0)
    ht = h.T
    xst = jnp.concatenate([sm_ref[...], dm_ref[...]], axis=1).T
    for c in range(64):
        a_ref[c * 128:(c + 1) * 128, :] = (
            ht[c:c + 1, :] * xst).astype(jnp.bfloat16)
    out = jnp.dot(eet_ref[...], a_ref[...], preferred_element_type=f32)
    xdt = jnp.concatenate([xst[MEMORY_DIM:, :], xst[:MEMORY_DIM, :]], axis=0)
    bbt = bbt_ref[...]
    bias_s = jnp.dot(bbt, xst, preferred_element_type=f32)
    bias_d = jnp.dot(bbt, xdt, preferred_element_type=f32)
    o_ref[0, :, :] = (out[0:MSG_DIM, :] + bias_s).T
    o_ref[1, :, :] = (out[MSG_DIM:, :] + bias_d).T


def _tc_messages(tf, edge_attr, src_m, dst_m, w_time, b_time,
                 en_w1, en_b1, en_w2, en_b2):
    eet_s = en_w2.reshape(64, 2 * MEMORY_DIM, MSG_DIM).transpose(2, 0, 1)
    eet_d = en_w2.reshape(64, 2, MEMORY_DIM, MSG_DIM)[:, ::-1].reshape(
        64, 2 * MEMORY_DIM, MSG_DIM).transpose(2, 0, 1)
    eet = jnp.concatenate([eet_s, eet_d], axis=0).reshape(
        2 * MSG_DIM, 2 * MEMORY_DIM * 64)
    bbt = en_b2.reshape(2 * MEMORY_DIM, MSG_DIM).T
    full = lambda shape: pl.BlockSpec(shape, lambda i: (0,) * len(shape))
    out = pl.pallas_call(
        _tc_msg_body,
        grid=(GRID1,),
        in_specs=[
            pl.BlockSpec((TB, 1), lambda i: (i, 0)),
            pl.BlockSpec((TB, EDGE_DIM), lambda i: (i, 0)),
            pl.BlockSpec((TB, MEMORY_DIM), lambda i: (i, 0)),
            pl.BlockSpec((TB, MEMORY_DIM), lambda i: (i, 0)),
            full((1, TIME_DIM)),
            full((1, TIME_DIM)),
            full((RAW_MSG_DIM, 64)),
            full((1, 64)),
            full((2 * MSG_DIM, 2 * MEMORY_DIM * 64)),
            full((MSG_DIM, 2 * MEMORY_DIM)),
        ],
        out_specs=pl.BlockSpec((2, TB, MSG_DIM), lambda i: (0, i, 0)),
        out_shape=jax.ShapeDtypeStruct((2, B, MSG_DIM), jnp.float32),
        scratch_shapes=[pltpu.VMEM((2 * MEMORY_DIM * 64, TB), jnp.bfloat16)],
    )(tf, edge_attr, src_m, dst_m, w_time, b_time, en_w1, en_b1,
      eet.astype(jnp.bfloat16), bbt)
    return out.reshape(2 * B, MSG_DIM)


def _build_last_pos_table(ids_full, tab_v, nb_v):
    iota = lax.iota(jnp.int32, 16)

    def body(it, carry):
        ids = ids_full[pl.ds(it * 16, 16)]
        p = it * 16 + iota
        key = ids * 2048 + p
        key_s = lax.sort(key)
        id_s = lax.shift_right_logical(key_s, 11)
        p_s = jnp.bitwise_and(key_s, 2047)
        nb_v[...] = id_s
        nxt = plsc.load_gather(nb_v, [jnp.minimum(iota + 1, 15)])
        keep = jnp.logical_or(id_s != nxt, iota == 15)
        plsc.store_scatter(tab_v, [id_s], p_s, mask=keep)
        return carry

    lax.fori_loop(0, B // 16, body, 0)


def _winner_chunk(ps_ref, pd_ref, j, t_v):
    ps = ps_ref[pl.ds(j * 16, 16)]
    pd = pd_ref[pl.ds(j * 16, 16)]
    tp = plsc.load_gather(t_v, [jnp.maximum(ps, 0)])
    td = plsc.load_gather(t_v, [jnp.maximum(pd, 0)])
    tp = jnp.where(ps >= 0, tp, -1)
    td = jnp.where(pd >= 0, td, -1)
    return jnp.where(td >= tp, pd + B, ps)


def _sc_agg_body(src_hbm, dst_hbm, t_hbm, neg1_hbm, msgs_hbm, o_as, o_ad,
                 ls_sh, ld_sh, ids_full, tab_v, nb_v, t_v,
                 idx_s, idx_d, ps_s, pd_s, ps_d, pd_d, w_s, w_d,
                 r_s, r_d, sem):
    cid = lax.axis_index("c")
    sid = lax.axis_index("s")
    wid = sid * NC + cid
    base = wid * BPW

    pltpu.sync_copy(t_hbm, t_v)
    pltpu.sync_copy(src_hbm.at[pl.ds(base, BPW)], idx_s)
    pltpu.sync_copy(dst_hbm.at[pl.ds(base, BPW)], idx_d)

    @pl.when(sid == 0)
    def _():
        pltpu.sync_copy(src_hbm, ids_full)
        pltpu.sync_copy(neg1_hbm, tab_v)
        _build_last_pos_table(ids_full, tab_v, nb_v)
        pltpu.sync_copy(tab_v, ls_sh)

    @pl.when(sid == 1)
    def _():
        pltpu.sync_copy(dst_hbm, ids_full)
        pltpu.sync_copy(neg1_hbm, tab_v)
        _build_last_pos_table(ids_full, tab_v, nb_v)
        pltpu.sync_copy(tab_v, ld_sh)

    plsc.subcore_barrier()

    c1 = pltpu.async_copy(ls_sh.at[idx_s], ps_s, sem)
    c2 = pltpu.async_copy(ld_sh.at[idx_s], pd_s, sem)
    c3 = pltpu.async_copy(ls_sh.at[idx_d], ps_d, sem)
    c4 = pltpu.async_copy(ld_sh.at[idx_d], pd_d, sem)
    c1.wait(); c2.wait(); c3.wait(); c4.wait()

    for j in range(BPW // 16):
        w_s[pl.ds(j * 16, 16)] = _winner_chunk(ps_s, pd_s, j, t_v)
        w_d[pl.ds(j * 16, 16)] = _winner_chunk(ps_d, pd_d, j, t_v)

    c1 = pltpu.async_copy(msgs_hbm.at[w_s], r_s, sem)
    c2 = pltpu.async_copy(msgs_hbm.at[w_d], r_d, sem)
    c1.wait(); c2.wait()
    pltpu.sync_copy(r_s, o_as.at[pl.ds(base, BPW)])
    pltpu.sync_copy(r_d, o_ad.at[pl.ds(base, BPW)])


def _sc_aggregate(src_i, dst_i, t_i, msgs):
    mesh = plsc.VectorSubcoreMesh(core_axis_name="c", subcore_axis_name="s")
    f32 = jnp.float32
    i32 = jnp.int32
    neg1 = jnp.full((NODES_PAD,), -1, i32)
    run = pl.kernel(
        _sc_agg_body,
        out_type=[
            jax.ShapeDtypeStruct((B, MSG_DIM), f32),
            jax.ShapeDtypeStruct((B, MSG_DIM), f32),
        ],
        mesh=mesh,
        scratch_types=[
            pltpu.VMEM_SHARED((NODES_PAD,), i32),
            pltpu.VMEM_SHARED((NODES_PAD,), i32),
            pltpu.VMEM((B,), i32),
            pltpu.VMEM((NODES_PAD,), i32),
            pltpu.VMEM((16,), i32),
            pltpu.VMEM((B,), i32),
            pltpu.VMEM((BPW,), i32),
            pltpu.VMEM((BPW,), i32),
            pltpu.VMEM((BPW,), i32),
            pltpu.VMEM((BPW,), i32),
            pltpu.VMEM((BPW,), i32),
            pltpu.VMEM((BPW,), i32),
            pltpu.VMEM((BPW,), i32),
            pltpu.VMEM((BPW,), i32),
            pltpu.VMEM((BPW, MSG_DIM), f32),
            pltpu.VMEM((BPW, MSG_DIM), f32),
            pltpu.SemaphoreType.DMA,
        ],
        compiler_params=pltpu.CompilerParams(use_tc_tiling_on_sc=False,
                                             needs_layout_passes=False),
    )
    return run(src_i, dst_i, t_i, neg1, msgs)


TB2 = 2048
GRID2 = B // TB2


def _tc_readout_body(as_ref, ad_ref, sm_ref, dm_ref, sst_ref, dst_ref,
                     se_ref, de_ref, sdy_ref, ddy_ref, ea_ref,
                     wi_ref, wh_ref, bi_ref, bh_ref,
                     m1_ref, mb1_ref, m2_ref, mb2_ref,
                     pws_ref, pwd_ref, pwe_ref, pb_ref, o_ref):
    f32 = jnp.float32
    agg = jnp.concatenate([as_ref[...], ad_ref[...]], axis=0)
    mem = jnp.concatenate([sm_ref[...], dm_ref[...]], axis=0)
    gi = jnp.dot(agg, wi_ref[...], preferred_element_type=f32) + bi_ref[...]
    gh = jnp.dot(mem, wh_ref[...], preferred_element_type=f32) + bh_ref[...]
    M = MEMORY_DIM
    r = jax.nn.sigmoid(gi[:, 0:M] + gh[:, 0:M])
    z = jax.nn.sigmoid(gi[:, M:2 * M] + gh[:, M:2 * M])
    n = jnp.tanh(gi[:, 2 * M:3 * M] + r * gh[:, 2 * M:3 * M])
    upd = (1.0 - z) * n + z * mem
    st = jnp.concatenate([sst_ref[...], dst_ref[...]], axis=0)
    emb = jnp.concatenate([se_ref[...], de_ref[...]], axis=0)
    dyn = jnp.concatenate([sdy_ref[...], ddy_ref[...]], axis=0)
    full = jnp.concatenate([upd, st, emb, dyn], axis=1).astype(jnp.bfloat16)
    h1 = jnp.maximum(
        jnp.dot(full, m1_ref[...], preferred_element_type=f32) + mb1_ref[...],
        0.0)
    e = jnp.dot(h1.astype(jnp.bfloat16), m2_ref[...],
                preferred_element_type=f32) + mb2_ref[...]
    es = e[0:TB2, :]
    ed = e[TB2:2 * TB2, :]
    pred = (jnp.dot(es, pws_ref[...], preferred_element_type=f32)
            + jnp.dot(ed, pwd_ref[...], preferred_element_type=f32)
            + jnp.dot(ea_ref[...], pwe_ref[...], preferred_element_type=f32)
            + pb_ref[...])
    o_ref[...] = pred


def _tc_readout(agg_s, agg_d, src_m, dst_m, src_static, dst_static,
                emb_s, emb_d, src_dynamic, dst_dynamic, edge_attr,
                gru_wi, gru_wh, gru_bi, gru_bh,
                mlp_w1, mlp_b1, mlp_w2, mlp_b2, pw_s, pw_d, pw_e, pred_b):
    f32 = jnp.float32
    full = lambda shape: pl.BlockSpec(shape, lambda i: (0,) * len(shape))
    row = lambda d: pl.BlockSpec((TB2, d), lambda i: (i, 0))
    return pl.pallas_call(
        _tc_readout_body,
        grid=(GRID2,),
        in_specs=[
            row(MSG_DIM), row(MSG_DIM),
            row(MEMORY_DIM), row(MEMORY_DIM),
            row(STATIC_DIM), row(STATIC_DIM),
            row(EMB_DIM), row(EMB_DIM),
            row(DYNAMIC_DIM), row(DYNAMIC_DIM),
            row(EDGE_DIM),
            full((MSG_DIM, 3 * MEMORY_DIM)),
            full((MEMORY_DIM, 3 * MEMORY_DIM)),
            full((1, 3 * MEMORY_DIM)),
            full((1, 3 * MEMORY_DIM)),
            full((MEMORY_DIM + STATIC_DIM + EMB_DIM + DYNAMIC_DIM, NODE_DIM)),
            full((1, NODE_DIM)),
            full((NODE_DIM, NODE_DIM)),
            full((1, NODE_DIM)),
            full((NODE_DIM, 1)),
            full((NODE_DIM, 1)),
            full((EDGE_DIM, 1)),
            full((1, 1)),
        ],
        out_specs=pl.BlockSpec((TB2, 1), lambda i: (i, 0)),
        out_shape=jax.ShapeDtypeStruct((B, 1), f32),
    )(agg_s, agg_d, src_m, dst_m, src_static, dst_static,
      emb_s, emb_d, src_dynamic, dst_dynamic, edge_attr,
      gru_wi, gru_wh, gru_bi, gru_bh,
      mlp_w1.astype(jnp.bfloat16), mlp_b1, mlp_w2.astype(jnp.bfloat16),
      mlp_b2, pw_s, pw_d, pw_e, pred_b)


def kernel(src, dst, t, edge_attr, src_static, dst_static, src_dynamic,
           dst_dynamic, memory, last_update, w_time, b_time, en_w1, en_b1,
           en_w2, en_b2, gru_wi, gru_wh, gru_bi, gru_bh, emb_table,
           mlp_w1, mlp_b1, mlp_w2, mlp_b2, pred_w, pred_b):
    i32 = jnp.int32
    f32 = jnp.float32
    src_i = src.astype(i32)
    dst_i = dst.astype(i32)
    t_i = t.astype(i32)

    src_m, dst_m, emb_s, emb_d = _sc_gather(memory, emb_table, src_i, dst_i)

    tf = t_i.astype(f32).reshape(B, 1)
    msgs = _tc_messages(tf, edge_attr, src_m, dst_m,
                        w_time.reshape(1, TIME_DIM),
                        b_time.reshape(1, TIME_DIM),
                        en_w1, en_b1.reshape(1, 64),
                        en_w2, en_b2.reshape(1, 2 * MEMORY_DIM * MSG_DIM))

    agg_s, agg_d = _sc_aggregate(src_i, dst_i, t_i, msgs)

    pred = _tc_readout(
        agg_s, agg_d, src_m, dst_m, src_static, dst_static,
        emb_s, emb_d, src_dynamic, dst_dynamic, edge_attr,
        gru_wi, gru_wh, gru_bi.reshape(1, 3 * MEMORY_DIM),
        gru_bh.reshape(1, 3 * MEMORY_DIM),
        mlp_w1, mlp_b1.reshape(1, NODE_DIM), mlp_w2,
        mlp_b2.reshape(1, NODE_DIM),
        pred_w[0:NODE_DIM], pred_w[NODE_DIM:2 * NODE_DIM],
        pred_w[2 * NODE_DIM:], pred_b.reshape(1, 1))
    return pred


# speedup vs baseline: 1.0134x; 1.0134x over previous
import functools

import jax
import jax.numpy as jnp
from jax import lax
from jax.experimental import pallas as pl
from jax.experimental.pallas import tpu as pltpu, tpu_sc as plsc

NUM_NODES = 10000
MEMORY_DIM = 64
MSG_DIM = 64
NODE_DIM = 256
EDGE_DIM = 16
TIME_DIM = 16
STATIC_DIM = 64
DYNAMIC_DIM = 64
EMB_DIM = 32
B = 2048
RAW_MSG_DIM = EDGE_DIM + TIME_DIM

NC = 2
NS = 16
NW = NC * NS
BPW = B // NW
NODES_PAD = 10016


def _sc_gather_body(mem_hbm, emb_hbm, src_hbm, dst_hbm,
                    o_sm, o_dm, o_es, o_ed,
                    idx_s, idx_d, r_sm, r_dm, r_es, r_ed, sem):
    wid = lax.axis_index("s") * NC + lax.axis_index("c")
    base = wid * BPW
    pltpu.sync_copy(src_hbm.at[pl.ds(base, BPW)], idx_s)
    pltpu.sync_copy(dst_hbm.at[pl.ds(base, BPW)], idx_d)
    c1 = pltpu.async_copy(mem_hbm.at[idx_s], r_sm, sem)
    c2 = pltpu.async_copy(mem_hbm.at[idx_d], r_dm, sem)
    c3 = pltpu.async_copy(emb_hbm.at[idx_s], r_es, sem)
    c4 = pltpu.async_copy(emb_hbm.at[idx_d], r_ed, sem)
    c1.wait(); c2.wait(); c3.wait(); c4.wait()
    pltpu.sync_copy(r_sm, o_sm.at[pl.ds(base, BPW)])
    pltpu.sync_copy(r_dm, o_dm.at[pl.ds(base, BPW)])
    pltpu.sync_copy(r_es, o_es.at[pl.ds(base, BPW)])
    pltpu.sync_copy(r_ed, o_ed.at[pl.ds(base, BPW)])


def _sc_gather(memory, emb_table, src_i, dst_i):
    mesh = plsc.VectorSubcoreMesh(core_axis_name="c", subcore_axis_name="s")
    f32 = jnp.float32
    run = pl.kernel(
        _sc_gather_body,
        out_type=[
            jax.ShapeDtypeStruct((B, MEMORY_DIM), f32),
            jax.ShapeDtypeStruct((B, MEMORY_DIM), f32),
            jax.ShapeDtypeStruct((B, EMB_DIM), f32),
            jax.ShapeDtypeStruct((B, EMB_DIM), f32),
        ],
        mesh=mesh,
        scratch_types=[
            pltpu.VMEM((BPW,), jnp.int32),
            pltpu.VMEM((BPW,), jnp.int32),
            pltpu.VMEM((BPW, MEMORY_DIM), f32),
            pltpu.VMEM((BPW, MEMORY_DIM), f32),
            pltpu.VMEM((BPW, EMB_DIM), f32),
            pltpu.VMEM((BPW, EMB_DIM), f32),
            pltpu.SemaphoreType.DMA,
        ],
        compiler_params=pltpu.CompilerParams(use_tc_tiling_on_sc=False),
    )
    return run(memory, emb_table, src_i, dst_i)


TB = 1024
GRID1 = B // TB


def _tc_msg_body(tf_ref, ea_ref, sm_ref, dm_ref, wt_ref, bt_ref,
                 w1_ref, b1_ref, eet_ref, bbt_ref, o_ref, a_ref):
    f32 = jnp.float32
    tf = tf_ref[...]
    te = jnp.cos(tf * wt_ref[...] + bt_ref[...])
    raw = jnp.concatenate([ea_ref[...], te], axis=1)
    h = jnp.maximum(
        jnp.dot(raw, w1_ref[...], preferred_element_type=f32)
        + b1_ref[...], 0.0)
    ht = h.T
    xst = jnp.concatenate([sm_ref[...], dm_ref[...]], axis=1).T
    for c in range(64):
        a_ref[c * 128:(c + 1) * 128, :] = (
            ht[c:c + 1, :] * xst).astype(jnp.bfloat16)
    out = jnp.dot(eet_ref[...], a_ref[...], preferred_element_type=f32)
    xdt = jnp.concatenate([xst[MEMORY_DIM:, :], xst[:MEMORY_DIM, :]], axis=0)
    bbt = bbt_ref[...]
    bias_s = jnp.dot(bbt, xst, preferred_element_type=f32)
    bias_d = jnp.dot(bbt, xdt, preferred_element_type=f32)
    o_ref[0, :, :] = (out[0:MSG_DIM, :] + bias_s).T
    o_ref[1, :, :] = (out[MSG_DIM:, :] + bias_d).T


def _tc_messages(tf, edge_attr, src_m, dst_m, w_time, b_time,
                 en_w1, en_b1, en_w2, en_b2):
    eet_s = en_w2.reshape(64, 2 * MEMORY_DIM, MSG_DIM).transpose(2, 0, 1)
    eet_d = en_w2.reshape(64, 2, MEMORY_DIM, MSG_DIM)[:, ::-1].reshape(
        64, 2 * MEMORY_DIM, MSG_DIM).transpose(2, 0, 1)
    eet = jnp.concatenate([eet_s, eet_d], axis=0).reshape(
        2 * MSG_DIM, 2 * MEMORY_DIM * 64)
    bbt = en_b2.reshape(2 * MEMORY_DIM, MSG_DIM).T
    full = lambda shape: pl.BlockSpec(shape, lambda i: (0,) * len(shape))
    out = pl.pallas_call(
        _tc_msg_body,
        grid=(GRID1,),
        in_specs=[
            pl.BlockSpec((TB, 1), lambda i: (i, 0)),
            pl.BlockSpec((TB, EDGE_DIM), lambda i: (i, 0)),
            pl.BlockSpec((TB, MEMORY_DIM), lambda i: (i, 0)),
            pl.BlockSpec((TB, MEMORY_DIM), lambda i: (i, 0)),
            full((1, TIME_DIM)),
            full((1, TIME_DIM)),
            full((RAW_MSG_DIM, 64)),
            full((1, 64)),
            full((2 * MSG_DIM, 2 * MEMORY_DIM * 64)),
            full((MSG_DIM, 2 * MEMORY_DIM)),
        ],
        out_specs=pl.BlockSpec((2, TB, MSG_DIM), lambda i: (0, i, 0)),
        out_shape=jax.ShapeDtypeStruct((2, B, MSG_DIM), jnp.float32),
        scratch_shapes=[pltpu.VMEM((2 * MEMORY_DIM * 64, TB), jnp.bfloat16)],
    )(tf, edge_attr, src_m, dst_m, w_time, b_time, en_w1, en_b1,
      eet.astype(jnp.bfloat16), bbt)
    return out.reshape(2 * B, MSG_DIM)


def _build_last_pos_table(ids_full, tab_v, nb_v):
    iota = lax.iota(jnp.int32, 16)

    def body(it, carry):
        ids = ids_full[pl.ds(it * 16, 16)]
        p = it * 16 + iota
        key = ids * 2048 + p
        key_s = lax.sort(key)
        id_s = lax.shift_right_logical(key_s, 11)
        p_s = jnp.bitwise_and(key_s, 2047)
        nb_v[...] = id_s
        nxt = plsc.load_gather(nb_v, [jnp.minimum(iota + 1, 15)])
        keep = jnp.logical_or(id_s != nxt, iota == 15)
        plsc.store_scatter(tab_v, [id_s], p_s, mask=keep)
        return carry

    lax.fori_loop(0, B // 16, body, 0)


def _winner_chunk(ps_ref, pd_ref, j, t_v):
    ps = ps_ref[pl.ds(j * 16, 16)]
    pd = pd_ref[pl.ds(j * 16, 16)]
    tp = plsc.load_gather(t_v, [jnp.maximum(ps, 0)])
    td = plsc.load_gather(t_v, [jnp.maximum(pd, 0)])
    tp = jnp.where(ps >= 0, tp, -1)
    td = jnp.where(pd >= 0, td, -1)
    return jnp.where(td >= tp, pd + B, ps)


def _sc_agg_body(src_hbm, dst_hbm, t_hbm, neg1_hbm, msgs_hbm, o_as, o_ad,
                 ls_sh, ld_sh, ids_full, tab_v, nb_v, t_v,
                 idx_s, idx_d, ps_s, pd_s, ps_d, pd_d, w_s, w_d,
                 r_s, r_d, sem):
    cid = lax.axis_index("c")
    sid = lax.axis_index("s")
    wid = sid * NC + cid
    base = wid * BPW

    pltpu.sync_copy(t_hbm, t_v)
    pltpu.sync_copy(src_hbm.at[pl.ds(base, BPW)], idx_s)
    pltpu.sync_copy(dst_hbm.at[pl.ds(base, BPW)], idx_d)

    @pl.when(sid == 0)
    def _():
        pltpu.sync_copy(src_hbm, ids_full)
        pltpu.sync_copy(neg1_hbm, tab_v)
        _build_last_pos_table(ids_full, tab_v, nb_v)
        pltpu.sync_copy(tab_v, ls_sh)

    @pl.when(sid == 1)
    def _():
        pltpu.sync_copy(dst_hbm, ids_full)
        pltpu.sync_copy(neg1_hbm, tab_v)
        _build_last_pos_table(ids_full, tab_v, nb_v)
        pltpu.sync_copy(tab_v, ld_sh)

    plsc.subcore_barrier()

    c1 = pltpu.async_copy(ls_sh.at[idx_s], ps_s, sem)
    c2 = pltpu.async_copy(ld_sh.at[idx_s], pd_s, sem)
    c3 = pltpu.async_copy(ls_sh.at[idx_d], ps_d, sem)
    c4 = pltpu.async_copy(ld_sh.at[idx_d], pd_d, sem)
    c1.wait(); c2.wait(); c3.wait(); c4.wait()

    for j in range(BPW // 16):
        w_s[pl.ds(j * 16, 16)] = _winner_chunk(ps_s, pd_s, j, t_v)
        w_d[pl.ds(j * 16, 16)] = _winner_chunk(ps_d, pd_d, j, t_v)

    c1 = pltpu.async_copy(msgs_hbm.at[w_s], r_s, sem)
    c2 = pltpu.async_copy(msgs_hbm.at[w_d], r_d, sem)
    c1.wait(); c2.wait()
    pltpu.sync_copy(r_s, o_as.at[pl.ds(base, BPW)])
    pltpu.sync_copy(r_d, o_ad.at[pl.ds(base, BPW)])


def _sc_aggregate(src_i, dst_i, t_i, msgs):
    mesh = plsc.VectorSubcoreMesh(core_axis_name="c", subcore_axis_name="s")
    f32 = jnp.float32
    i32 = jnp.int32
    neg1 = jnp.full((NODES_PAD,), -1, i32)
    run = pl.kernel(
        _sc_agg_body,
        out_type=[
            jax.ShapeDtypeStruct((B, MSG_DIM), f32),
            jax.ShapeDtypeStruct((B, MSG_DIM), f32),
        ],
        mesh=mesh,
        scratch_types=[
            pltpu.VMEM_SHARED((NODES_PAD,), i32),
            pltpu.VMEM_SHARED((NODES_PAD,), i32),
            pltpu.VMEM((B,), i32),
            pltpu.VMEM((NODES_PAD,), i32),
            pltpu.VMEM((16,), i32),
            pltpu.VMEM((B,), i32),
            pltpu.VMEM((BPW,), i32),
            pltpu.VMEM((BPW,), i32),
            pltpu.VMEM((BPW,), i32),
            pltpu.VMEM((BPW,), i32),
            pltpu.VMEM((BPW,), i32),
            pltpu.VMEM((BPW,), i32),
            pltpu.VMEM((BPW,), i32),
            pltpu.VMEM((BPW,), i32),
            pltpu.VMEM((BPW, MSG_DIM), f32),
            pltpu.VMEM((BPW, MSG_DIM), f32),
            pltpu.SemaphoreType.DMA,
        ],
        compiler_params=pltpu.CompilerParams(use_tc_tiling_on_sc=False,
                                             needs_layout_passes=False),
    )
    return run(src_i, dst_i, t_i, neg1, msgs)


TB2 = 1024
GRID2 = B // TB2


def _tc_readout_body(as_ref, ad_ref, sm_ref, dm_ref, sst_ref, dst_ref,
                     se_ref, de_ref, sdy_ref, ddy_ref, ea_ref,
                     wi_ref, wh_ref, bi_ref, bh_ref,
                     m1_ref, mb1_ref, m2_ref, mb2_ref,
                     pws_ref, pwd_ref, pwe_ref, pb_ref, o_ref):
    f32 = jnp.float32
    agg = jnp.concatenate([as_ref[...], ad_ref[...]], axis=0)
    mem = jnp.concatenate([sm_ref[...], dm_ref[...]], axis=0)
    gi = jnp.dot(agg, wi_ref[...], preferred_element_type=f32) + bi_ref[...]
    gh = jnp.dot(mem, wh_ref[...], preferred_element_type=f32) + bh_ref[...]
    M = MEMORY_DIM
    r = jax.nn.sigmoid(gi[:, 0:M] + gh[:, 0:M])
    z = jax.nn.sigmoid(gi[:, M:2 * M] + gh[:, M:2 * M])
    n = jnp.tanh(gi[:, 2 * M:3 * M] + r * gh[:, 2 * M:3 * M])
    upd = (1.0 - z) * n + z * mem
    st = jnp.concatenate([sst_ref[...], dst_ref[...]], axis=0)
    emb = jnp.concatenate([se_ref[...], de_ref[...]], axis=0)
    dyn = jnp.concatenate([sdy_ref[...], ddy_ref[...]], axis=0)
    full = jnp.concatenate([upd, st, emb, dyn], axis=1).astype(jnp.bfloat16)
    h1 = jnp.maximum(
        jnp.dot(full, m1_ref[...], preferred_element_type=f32) + mb1_ref[...],
        0.0)
    e = jnp.dot(h1.astype(jnp.bfloat16), m2_ref[...],
                preferred_element_type=f32) + mb2_ref[...]
    es = e[0:TB2, :]
    ed = e[TB2:2 * TB2, :]
    pred = (jnp.dot(es, pws_ref[...], preferred_element_type=f32)
            + jnp.dot(ed, pwd_ref[...], preferred_element_type=f32)
            + jnp.dot(ea_ref[...], pwe_ref[...], preferred_element_type=f32)
            + pb_ref[...])
    o_ref[...] = pred


def _tc_readout(agg_s, agg_d, src_m, dst_m, src_static, dst_static,
                emb_s, emb_d, src_dynamic, dst_dynamic, edge_attr,
                gru_wi, gru_wh, gru_bi, gru_bh,
                mlp_w1, mlp_b1, mlp_w2, mlp_b2, pw_s, pw_d, pw_e, pred_b):
    f32 = jnp.float32
    full = lambda shape: pl.BlockSpec(shape, lambda i: (0,) * len(shape))
    row = lambda d: pl.BlockSpec((TB2, d), lambda i: (i, 0))
    return pl.pallas_call(
        _tc_readout_body,
        grid=(GRID2,),
        in_specs=[
            row(MSG_DIM), row(MSG_DIM),
            row(MEMORY_DIM), row(MEMORY_DIM),
            row(STATIC_DIM), row(STATIC_DIM),
            row(EMB_DIM), row(EMB_DIM),
            row(DYNAMIC_DIM), row(DYNAMIC_DIM),
            row(EDGE_DIM),
            full((MSG_DIM, 3 * MEMORY_DIM)),
            full((MEMORY_DIM, 3 * MEMORY_DIM)),
            full((1, 3 * MEMORY_DIM)),
            full((1, 3 * MEMORY_DIM)),
            full((MEMORY_DIM + STATIC_DIM + EMB_DIM + DYNAMIC_DIM, NODE_DIM)),
            full((1, NODE_DIM)),
            full((NODE_DIM, NODE_DIM)),
            full((1, NODE_DIM)),
            full((NODE_DIM, 1)),
            full((NODE_DIM, 1)),
            full((EDGE_DIM, 1)),
            full((1, 1)),
        ],
        out_specs=pl.BlockSpec((TB2, 1), lambda i: (i, 0)),
        out_shape=jax.ShapeDtypeStruct((B, 1), f32),
    )(agg_s, agg_d, src_m, dst_m, src_static, dst_static,
      emb_s, emb_d, src_dynamic, dst_dynamic, edge_attr,
      gru_wi, gru_wh, gru_bi, gru_bh,
      mlp_w1.astype(jnp.bfloat16), mlp_b1, mlp_w2.astype(jnp.bfloat16),
      mlp_b2, pw_s, pw_d, pw_e, pred_b)


def kernel(src, dst, t, edge_attr, src_static, dst_static, src_dynamic,
           dst_dynamic, memory, last_update, w_time, b_time, en_w1, en_b1,
           en_w2, en_b2, gru_wi, gru_wh, gru_bi, gru_bh, emb_table,
           mlp_w1, mlp_b1, mlp_w2, mlp_b2, pred_w, pred_b):
    i32 = jnp.int32
    f32 = jnp.float32
    src_i = src.astype(i32)
    dst_i = dst.astype(i32)
    t_i = t.astype(i32)

    src_m, dst_m, emb_s, emb_d = _sc_gather(memory, emb_table, src_i, dst_i)

    tf = t_i.astype(f32).reshape(B, 1)
    msgs = _tc_messages(tf, edge_attr, src_m, dst_m,
                        w_time.reshape(1, TIME_DIM),
                        b_time.reshape(1, TIME_DIM),
                        en_w1, en_b1.reshape(1, 64),
                        en_w2, en_b2.reshape(1, 2 * MEMORY_DIM * MSG_DIM))

    agg_s, agg_d = _sc_aggregate(src_i, dst_i, t_i, msgs)

    pred = _tc_readout(
        agg_s, agg_d, src_m, dst_m, src_static, dst_static,
        emb_s, emb_d, src_dynamic, dst_dynamic, edge_attr,
        gru_wi, gru_wh, gru_bi.reshape(1, 3 * MEMORY_DIM),
        gru_bh.reshape(1, 3 * MEMORY_DIM),
        mlp_w1, mlp_b1.reshape(1, NODE_DIM), mlp_w2,
        mlp_b2.reshape(1, NODE_DIM),
        pred_w[0:NODE_DIM], pred_w[NODE_DIM:2 * NODE_DIM],
        pred_w[2 * NODE_DIM:], pred_b.reshape(1, 1))
    return pred
